# baseline (speedup 1.0000x reference)
import functools

import jax
import jax.numpy as jnp
from jax import lax
from jax.experimental import pallas as pl
from jax.experimental.pallas import tpu as pltpu
from jax.experimental.pallas import tpu_sc as plsc

B, L, D = 1024, 200, 128
MAX_POS = 10000
LANE = 16
_info = plsc.get_sparse_core_info()
NC, NS = _info.num_cores, _info.num_subcores
NW = NC * NS
ROWS_PER_W = B // NW
NHALF = 2 * ROWS_PER_W
CH = (104, 96)
OFF = (0, 104)

_mesh = plsc.VectorSubcoreMesh(core_axis_name="c", subcore_axis_name="s")

_GATHER_DNUMS = lax.GatherDimensionNumbers(
    offset_dims=(), collapsed_slice_dims=(0,), start_index_map=(0,))


def _lane_permute(x, perm):
    return lax.gather(
        x, perm[:, None], _GATHER_DNUMS, slice_sizes=(1,),
        mode=lax.GatherScatterMode.PROMISE_IN_BOUNDS)


@functools.partial(
    pl.kernel,
    out_type=jax.ShapeDtypeStruct((B, L, D), jnp.float32),
    mesh=_mesh,
    scratch_types=[
        pltpu.VMEM((ROWS_PER_W, L), jnp.int32),
        pltpu.VMEM((L,), jnp.int32),
        pltpu.VMEM((L,), jnp.int32),
        pltpu.VMEM((CH[0], D), jnp.float32),
        pltpu.VMEM((CH[0], D), jnp.float32),
        pltpu.VMEM((CH[0], D), jnp.float32),
        pltpu.VMEM_SHARED((MAX_POS, D), jnp.float32),
        pltpu.SemaphoreType.DMA,
        pltpu.SemaphoreType.DMA,
        pltpu.SemaphoreType.DMA,
        pltpu.SemaphoreType.DMA,
        pltpu.SemaphoreType.DMA,
        pltpu.SemaphoreType.DMA,
        pltpu.SemaphoreType.DMA,
    ],
)
def _pe_kernel(vco_hbm, table_hbm, out_hbm, idx_v, rel0_v, rel1_v,
               bufa, bufb, bufc, table_sh,
               gs0, gs1, gs2, os0, os1, os2, stage_sem):
    wid = lax.axis_index("s") * NC + lax.axis_index("c")
    base = wid * ROWS_PER_W
    sid = lax.axis_index("s")
    stage_rows = (MAX_POS // NS) // 8 * 8
    tail = MAX_POS - NS * stage_rows
    stage_cp = pltpu.async_copy(
        table_hbm.at[pl.ds(sid * stage_rows, stage_rows)],
        table_sh.at[pl.ds(sid * stage_rows, stage_rows)], stage_sem)

    @pl.when(sid == 0)
    def _stage_tail():
        pltpu.sync_copy(table_hbm.at[pl.ds(NS * stage_rows, tail)],
                        table_sh.at[pl.ds(NS * stage_rows, tail)])

    pltpu.sync_copy(vco_hbm.at[pl.ds(base, ROWS_PER_W)], idx_v)

    rels = (rel0_v, rel1_v)
    bufs = (bufa, bufb, bufc)
    gsems = (gs0, gs1, gs2)
    osems = (os0, os1, os2)

    def compute_rel(r, p):
        m = idx_v[r, pl.ds(0, LANE)]
        for k in range(1, L // LANE):
            m = jnp.minimum(m, idx_v[r, pl.ds(k * LANE, LANE)])
        m = jnp.minimum(m, idx_v[r, pl.ds(L - LANE, LANE)])
        lanes = lax.iota(jnp.int32, LANE)
        for sh in (8, 4, 2, 1):
            perm = lax.rem(lanes + sh, LANE)
            m = jnp.minimum(m, _lane_permute(m, perm))
        rel_v = rels[p]
        for k in range(L // LANE):
            rel_v[pl.ds(k * LANE, LANE)] = jnp.abs(
                idx_v[r, pl.ds(k * LANE, LANE)] - m)
        rel_v[pl.ds(L - LANE, LANE)] = jnp.abs(
            idx_v[r, pl.ds(L - LANE, LANE)] - m)

    def g_desc(r, p, h, b):
        return pltpu.make_async_copy(
            table_sh.at[rels[p].at[pl.ds(OFF[h], CH[h])]],
            bufs[b].at[pl.ds(0, CH[h])], gsems[b])

    def o_desc(r, h, b):
        return pltpu.make_async_copy(
            bufs[b].at[pl.ds(0, CH[h])],
            out_hbm.at[base + r, pl.ds(OFF[h], CH[h])], osems[b])

    def fire_g(r, p, h, b):
        pltpu.async_copy(
            table_sh.at[rels[p].at[pl.ds(OFF[h], CH[h])]],
            bufs[b].at[pl.ds(0, CH[h])], gsems[b])

    def fire_o(r, h, b):
        pltpu.async_copy(
            bufs[b].at[pl.ds(0, CH[h])],
            out_hbm.at[base + r, pl.ds(OFF[h], CH[h])], osems[b])

    def do_step(c_mod12, row_of, first=False, fire_next=True):
        j = c_mod12
        hc, bc = j % 2, j % 3
        pc = (j // 2) % 2
        g_desc(row_of(j), pc, hc, bc).wait()
        fire_o(row_of(j), hc, bc)
        if not first:
            hp, bp = (j - 1) % 2, (j - 1) % 3
            o_desc(row_of(j - 1), hp, bp).wait()
        if fire_next:
            hn, bn = (j + 2) % 2, (j + 2) % 3
            pn = ((j + 2) // 2) % 2
            if j % 2 == 0:
                compute_rel(row_of(j + 2), pn)
            fire_g(row_of(j + 2), pn, hn, bn)

    compute_rel(0, 0)
    stage_cp.wait()
    plsc.subcore_barrier()
    fire_g(0, 0, 0, 0)
    fire_g(0, 0, 1, 1)
    do_step(0, lambda j: j // 2, first=True)
    do_step(1, lambda j: j // 2)

    def step(s, carry):
        base_c = 12 * s + 2
        for k in range(12):
            c = base_c + k
            jmod = (2 + k) % 12

            def row_of(j, c=c, jmod=jmod):
                return (c + (j - jmod)) // 2

            do_step(jmod, row_of)
        return carry

    lax.fori_loop(0, (NHALF - 4) // 12, step, 0)

    last = ROWS_PER_W - 1

    def row_of_62(j):
        return (62 + (j - (62 % 12))) // 2

    def row_of_63(j):
        return (63 + (j - (63 % 12))) // 2

    do_step(62 % 12, row_of_62, fire_next=False)
    do_step(63 % 12, row_of_63, fire_next=False)
    fire_o(last, 1, 63 % 3)
    o_desc(last, 1, 63 % 3).wait()


def kernel(visit_concept_orders, pos_encoding):
    return _pe_kernel(visit_concept_orders.astype(jnp.int32), pos_encoding)


# speedup vs baseline: 1.0736x; 1.0736x over previous
import functools

import jax
import jax.numpy as jnp
from jax import lax
from jax.experimental import pallas as pl
from jax.experimental.pallas import tpu as pltpu
from jax.experimental.pallas import tpu_sc as plsc

B, L, D = 1024, 200, 128
MAX_POS = 10000
LANE = 16
_info = plsc.get_sparse_core_info()
NC, NS = _info.num_cores, _info.num_subcores
NW = NC * NS
ROWS_PER_W = B // NW
NHALF = 2 * ROWS_PER_W
CH = (104, 96)
OFF = (0, 104)

_mesh = plsc.VectorSubcoreMesh(core_axis_name="c", subcore_axis_name="s")

_GATHER_DNUMS = lax.GatherDimensionNumbers(
    offset_dims=(), collapsed_slice_dims=(0,), start_index_map=(0,))


def _lane_permute(x, perm):
    return lax.gather(
        x, perm[:, None], _GATHER_DNUMS, slice_sizes=(1,),
        mode=lax.GatherScatterMode.PROMISE_IN_BOUNDS)


@functools.partial(
    pl.kernel,
    out_type=jax.ShapeDtypeStruct((B, L, D), jnp.float32),
    mesh=_mesh,
    scratch_types=[
        pltpu.VMEM((ROWS_PER_W, L), jnp.int32),
        pltpu.VMEM((L,), jnp.int32),
        pltpu.VMEM((L,), jnp.int32),
        pltpu.VMEM((CH[0], D), jnp.float32),
        pltpu.VMEM((CH[0], D), jnp.float32),
        pltpu.VMEM((CH[0], D), jnp.float32),
        pltpu.VMEM_SHARED((MAX_POS, D), jnp.float32),
        pltpu.SemaphoreType.DMA,
        pltpu.SemaphoreType.DMA,
        pltpu.SemaphoreType.DMA,
        pltpu.SemaphoreType.DMA,
        pltpu.SemaphoreType.DMA,
        pltpu.SemaphoreType.DMA,
        pltpu.SemaphoreType.DMA,
    ],
)
def _pe_kernel(vco_hbm, table_hbm, out_hbm, idx_v, rel0_v, rel1_v,
               bufa, bufb, bufc, table_sh,
               gs0, gs1, gs2, os0, os1, os2, stage_sem):
    wid = lax.axis_index("s") * NC + lax.axis_index("c")
    base = wid * ROWS_PER_W
    sid = lax.axis_index("s")
    stage_rows = (MAX_POS // NS) // 8 * 8
    tail = MAX_POS - NS * stage_rows
    stage_cp = pltpu.async_copy(
        table_hbm.at[pl.ds(sid * stage_rows, stage_rows)],
        table_sh.at[pl.ds(sid * stage_rows, stage_rows)], stage_sem)

    @pl.when(sid == 0)
    def _stage_tail():
        pltpu.sync_copy(table_hbm.at[pl.ds(NS * stage_rows, tail)],
                        table_sh.at[pl.ds(NS * stage_rows, tail)])

    pltpu.sync_copy(vco_hbm.at[pl.ds(base, ROWS_PER_W)], idx_v)

    rels = (rel0_v, rel1_v)
    bufs = (bufa, bufb, bufc)
    gsems = (gs0, gs1, gs2)
    osems = (os0, os1, os2)

    def compute_rel(r, p):
        m = idx_v[r, pl.ds(0, LANE)]
        for k in range(1, L // LANE):
            m = jnp.minimum(m, idx_v[r, pl.ds(k * LANE, LANE)])
        m = jnp.minimum(m, idx_v[r, pl.ds(L - LANE, LANE)])
        lanes = lax.iota(jnp.int32, LANE)
        for sh in (8, 4, 2, 1):
            perm = lax.rem(lanes + sh, LANE)
            m = jnp.minimum(m, _lane_permute(m, perm))
        rel_v = rels[p]
        for k in range(L // LANE):
            rel_v[pl.ds(k * LANE, LANE)] = jnp.abs(
                idx_v[r, pl.ds(k * LANE, LANE)] - m)
        rel_v[pl.ds(L - LANE, LANE)] = jnp.abs(
            idx_v[r, pl.ds(L - LANE, LANE)] - m)

    def g_desc(r, p, h, b):
        return pltpu.make_async_copy(
            table_sh.at[rels[p].at[pl.ds(OFF[h], CH[h])]],
            bufs[b].at[pl.ds(0, CH[h])], gsems[b])

    def o_desc(r, h, b):
        return pltpu.make_async_copy(
            bufs[b].at[pl.ds(0, CH[h])],
            out_hbm.at[base + r, pl.ds(OFF[h], CH[h])], osems[b])

    def fire_g(r, p, h, b):
        pltpu.async_copy(
            table_sh.at[rels[p].at[pl.ds(OFF[h], CH[h])]],
            bufs[b].at[pl.ds(0, CH[h])], gsems[b])

    def fire_o(r, h, b):
        pltpu.async_copy(
            bufs[b].at[pl.ds(0, CH[h])],
            out_hbm.at[base + r, pl.ds(OFF[h], CH[h])], osems[b])

    def do_step(c_mod12, row_of, first=False, fire_next=True):
        j = c_mod12
        hc, bc = j % 2, j % 3
        pc = (j // 2) % 2
        g_desc(row_of(j), pc, hc, bc).wait()
        fire_o(row_of(j), hc, bc)
        if not first:
            hp, bp = (j - 1) % 2, (j - 1) % 3
            o_desc(row_of(j - 1), hp, bp).wait()
        if fire_next:
            hn, bn = (j + 2) % 2, (j + 2) % 3
            pn = ((j + 2) // 2) % 2
            if j % 2 == 0:
                compute_rel(row_of(j + 2), pn)
            fire_g(row_of(j + 2), pn, hn, bn)

    compute_rel(0, 0)
    stage_cp.wait()
    plsc.subcore_barrier()
    fire_g(0, 0, 0, 0)
    fire_g(0, 0, 1, 1)
    do_step(0, lambda j: j // 2, first=True)
    do_step(1, lambda j: j // 2)

    def step(s, carry):
        base_c = 12 * s + 2
        for k in range(12):
            c = base_c + k
            jmod = (2 + k) % 12

            def row_of(j, c=c, jmod=jmod):
                return (c + (j - jmod)) // 2

            do_step(jmod, row_of)
        return carry

    lax.fori_loop(0, (NHALF - 4) // 12, step, 0)

    last = ROWS_PER_W - 1

    def row_of_62(j):
        return (62 + (j - (62 % 12))) // 2

    def row_of_63(j):
        return (63 + (j - (63 % 12))) // 2

    do_step(62 % 12, row_of_62, fire_next=False)
    do_step(63 % 12, row_of_63, fire_next=False)
    o_desc(last, 1, 63 % 3).wait()


def kernel(visit_concept_orders, pos_encoding):
    return _pe_kernel(visit_concept_orders.astype(jnp.int32), pos_encoding)
